# Initial kernel scaffold; baseline (speedup 1.0000x reference)
import jax
import jax.numpy as jnp
from jax.experimental import pallas as pl


def kernel(x, edge_index, batch, W1, a1_src, a1_dst, b1, W2, a2_src, a2_dst, b2, W_mu, b_mu, W_sigma, b_sigma):
    raise NotImplementedError("write your pallas kernel here")


# speedup vs baseline: 26.8893x; 26.8893x over previous
import functools

import jax
import jax.numpy as jnp
from jax import lax
from jax.experimental import pallas as pl
from jax.experimental.pallas import tpu as pltpu
from jax.experimental.pallas import tpu_sc as plsc

N = 10000
E = 320000
D_IN = 128
D_H = 64
NGRAPH = 32
SEQ_OUT = 12
OUT_DIM = 14

RB = 128
NP = 10240
GRID = NP // RB
DA = 80
NT = 32
C = 128
NCH = 80
ET = NCH * C
EP = NT * ET
PAD_NODE = N
STRIPE = NP // 16


def _dense_in_body(x_ref, w_ref, a_ref, haug_ref, av_ref):
    h = jnp.dot(x_ref[...], w_ref[...], preferred_element_type=jnp.float32)
    haug_ref[...] = jnp.concatenate(
        [h, jnp.ones((RB, 1), jnp.float32),
         jnp.zeros((RB, DA - D_H - 1), jnp.float32)], axis=1)
    av_ref[...] = jnp.dot(h, a_ref[...], preferred_element_type=jnp.float32)


_dense_in = pl.pallas_call(
    _dense_in_body,
    grid=(GRID,),
    in_specs=[
        pl.BlockSpec((RB, D_IN), lambda i: (i, 0)),
        pl.BlockSpec((D_IN, D_H), lambda i: (0, 0)),
        pl.BlockSpec((D_H, 128), lambda i: (0, 0)),
    ],
    out_specs=[
        pl.BlockSpec((RB, DA), lambda i: (i, 0)),
        pl.BlockSpec((RB, 128), lambda i: (i, 0)),
    ],
    out_shape=[
        jax.ShapeDtypeStruct((NP, DA), jnp.float32),
        jax.ShapeDtypeStruct((NP, 128), jnp.float32),
    ],
)


def _finish_block(acc_ref, haug_ref, av_ref, b_ref):
    acc = acc_ref[0] + acc_ref[1]
    h = haug_ref[...][:, :D_H]
    es = av_ref[...][:, 0:1] + av_ref[...][:, 1:2]
    ws = jnp.exp(jnp.where(es >= 0, es, es * 0.2))
    den = acc[:, D_H:D_H + 1] + ws
    out = (acc[:, :D_H] + ws * h) / den + b_ref[...]
    return jnp.maximum(out, 0.0)


def _finish_dense_body(acc_ref, haug_ref, av_ref, b_ref, w_ref, a_ref,
                       haug2_ref, av2_ref):
    out1 = _finish_block(acc_ref, haug_ref, av_ref, b_ref)
    h2 = jnp.dot(out1, w_ref[...], preferred_element_type=jnp.float32)
    haug2_ref[...] = jnp.concatenate(
        [h2, jnp.ones((RB, 1), jnp.float32),
         jnp.zeros((RB, DA - D_H - 1), jnp.float32)], axis=1)
    av2_ref[...] = jnp.dot(h2, a_ref[...], preferred_element_type=jnp.float32)


_finish_dense = pl.pallas_call(
    _finish_dense_body,
    grid=(GRID,),
    in_specs=[
        pl.BlockSpec((2, RB, DA), lambda i: (0, i, 0)),
        pl.BlockSpec((RB, DA), lambda i: (i, 0)),
        pl.BlockSpec((RB, 128), lambda i: (i, 0)),
        pl.BlockSpec((1, D_H), lambda i: (0, 0)),
        pl.BlockSpec((D_H, D_H), lambda i: (0, 0)),
        pl.BlockSpec((D_H, 128), lambda i: (0, 0)),
    ],
    out_specs=[
        pl.BlockSpec((RB, DA), lambda i: (i, 0)),
        pl.BlockSpec((RB, 128), lambda i: (i, 0)),
    ],
    out_shape=[
        jax.ShapeDtypeStruct((NP, DA), jnp.float32),
        jax.ShapeDtypeStruct((NP, 128), jnp.float32),
    ],
)


def _finish_pool_body(acc_ref, haug_ref, av_ref, b_ref, batch_ref,
                      wmu_ref, bmu_ref, wsg_ref, bsg_ref,
                      mu_ref, sg_ref, g_acc, c_acc):
    i = pl.program_id(0)

    @pl.when(i == 0)
    def _():
        g_acc[...] = jnp.zeros_like(g_acc)
        c_acc[...] = jnp.zeros_like(c_acc)

    out2 = _finish_block(acc_ref, haug_ref, av_ref, b_ref)
    bt = batch_ref[0]
    gids = lax.broadcasted_iota(jnp.int32, (NGRAPH, 1), 0)
    oht = (bt == gids).astype(jnp.float32)
    g_acc[...] += jnp.dot(oht, out2, preferred_element_type=jnp.float32)
    c_acc[...] += jnp.sum(oht, axis=1, keepdims=True)

    @pl.when(i == GRID - 1)
    def _():
        g = g_acc[...] / jnp.maximum(c_acc[...], 1.0)
        mu_ref[...] = (jnp.dot(g, wmu_ref[...],
                               preferred_element_type=jnp.float32)
                       + bmu_ref[...])
        z = (jnp.dot(g, wsg_ref[...], preferred_element_type=jnp.float32)
             + bsg_ref[...])
        sg_ref[...] = jnp.maximum(z, 0.0) + jnp.log1p(jnp.exp(-jnp.abs(z)))


_finish_pool = pl.pallas_call(
    _finish_pool_body,
    grid=(GRID,),
    in_specs=[
        pl.BlockSpec((2, RB, DA), lambda i: (0, i, 0)),
        pl.BlockSpec((RB, DA), lambda i: (i, 0)),
        pl.BlockSpec((RB, 128), lambda i: (i, 0)),
        pl.BlockSpec((1, D_H), lambda i: (0, 0)),
        pl.BlockSpec((1, 1, RB), lambda i: (i, 0, 0)),
        pl.BlockSpec((D_H, SEQ_OUT * OUT_DIM), lambda i: (0, 0)),
        pl.BlockSpec((1, SEQ_OUT * OUT_DIM), lambda i: (0, 0)),
        pl.BlockSpec((D_H, SEQ_OUT * OUT_DIM), lambda i: (0, 0)),
        pl.BlockSpec((1, SEQ_OUT * OUT_DIM), lambda i: (0, 0)),
    ],
    out_specs=[
        pl.BlockSpec((NGRAPH, SEQ_OUT * OUT_DIM), lambda i: (0, 0)),
        pl.BlockSpec((NGRAPH, SEQ_OUT * OUT_DIM), lambda i: (0, 0)),
    ],
    out_shape=[
        jax.ShapeDtypeStruct((NGRAPH, SEQ_OUT * OUT_DIM), jnp.float32),
        jax.ShapeDtypeStruct((NGRAPH, SEQ_OUT * OUT_DIM), jnp.float32),
    ],
    scratch_shapes=[
        pltpu.VMEM((NGRAPH, D_H), jnp.float32),
        pltpu.VMEM((NGRAPH, 1), jnp.float32),
    ],
)


def _sc_edge_body(h_hbm, as_hbm, ad_hbm, src_hbm, dst_hbm, acc_out,
                  acc_sh, as_v, ad_v, src_v, dst_v, w_v,
                  rows_a, rows_b, zrow_v, sem_a, sem_b):
    c = lax.axis_index("c")
    s = lax.axis_index("s")
    wid = s * 2 + c

    pltpu.sync_copy(as_hbm, as_v)
    pltpu.sync_copy(ad_hbm, ad_v)
    pltpu.sync_copy(src_hbm.at[wid], src_v)
    pltpu.sync_copy(dst_hbm.at[wid], dst_v)

    zv = jnp.zeros((16,), jnp.float32)

    def _zrow(r, carry):
        for q in range(DA // 16):
            zrow_v[r, pl.ds(q * 16, 16)] = zv
        return carry

    lax.fori_loop(0, C, _zrow, 0)
    for k in range(STRIPE // C):
        pltpu.sync_copy(zrow_v, acc_sh.at[pl.ds(s * STRIPE + k * C, C)])
    plsc.subcore_barrier()

    def _half(jj, rows, sem, rows_next, sem_next):
        @pl.when(jj + 1 < NCH)
        def _():
            pltpu.async_copy(h_hbm.at[src_v.at[jj + 1]], rows_next, sem_next)

        for g in range(C // 16):
            sv = src_v[jj, pl.ds(g * 16, 16)]
            dv = dst_v[jj, pl.ds(g * 16, 16)]
            e = plsc.load_gather(as_v, [sv]) + plsc.load_gather(ad_v, [dv])
            e = jnp.where(e >= 0, e, e * jnp.float32(0.2))
            w_v[pl.ds(g * 16, 16)] = jnp.exp(e)

        pltpu.make_async_copy(h_hbm.at[src_v.at[jj]], rows, sem).wait()

        def _scale(t, carry):
            wv = w_v[pl.ds(t * 16, 16)]
            for u in range(16):
                ei = t * 16 + u
                wsc = wv[u]
                for q in range(DA // 16):
                    rows[ei, pl.ds(q * 16, 16)] = (
                        rows[ei, pl.ds(q * 16, 16)] * wsc)
            return carry

        lax.fori_loop(0, C // 16, _scale, 0)
        pltpu.sync_copy(rows, acc_sh.at[dst_v.at[jj]], add=True)

    pltpu.async_copy(h_hbm.at[src_v.at[0]], rows_a, sem_a)

    def _pair(k, carry):
        _half(k * 2, rows_a, sem_a, rows_b, sem_b)
        _half(k * 2 + 1, rows_b, sem_b, rows_a, sem_a)
        return carry

    lax.fori_loop(0, NCH // 2, _pair, 0)
    plsc.subcore_barrier()

    for k in range(STRIPE // C):
        off = s * STRIPE + k * C
        pltpu.sync_copy(acc_sh.at[pl.ds(off, C)],
                        acc_out.at[c, pl.ds(off, C)])


_sc_edge = pl.kernel(
    _sc_edge_body,
    out_type=jax.ShapeDtypeStruct((2, NP, DA), jnp.float32),
    mesh=plsc.VectorSubcoreMesh(core_axis_name="c", subcore_axis_name="s",
                                num_cores=2, num_subcores=16),
    compiler_params=pltpu.CompilerParams(needs_layout_passes=False,
                                         use_tc_tiling_on_sc=False),
    scratch_types=[
        pltpu.VMEM_SHARED((NP, DA), jnp.float32),
        pltpu.VMEM((NP,), jnp.float32),
        pltpu.VMEM((NP,), jnp.float32),
        pltpu.VMEM((NCH, C), jnp.int32),
        pltpu.VMEM((NCH, C), jnp.int32),
        pltpu.VMEM((C,), jnp.float32),
        pltpu.VMEM((C, DA), jnp.float32),
        pltpu.VMEM((C, DA), jnp.float32),
        pltpu.VMEM((C, DA), jnp.float32),
        pltpu.SemaphoreType.DMA,
        pltpu.SemaphoreType.DMA,
    ],
)


@jax.jit
def _run(x, edge_index, batch, W1, a1_src, a1_dst, b1,
         W2, a2_src, a2_dst, b2, W_mu, b_mu, W_sigma, b_sigma):
    f32 = jnp.float32
    x_pad = jnp.pad(x, ((0, NP - N), (0, 0)))
    A1 = jnp.zeros((D_H, 128), f32).at[:, 0].set(a1_src).at[:, 1].set(a1_dst)
    A2 = jnp.zeros((D_H, 128), f32).at[:, 0].set(a2_src).at[:, 1].set(a2_dst)
    pad_idx = jnp.full((EP - E,), PAD_NODE, jnp.int32)
    srcR = jnp.concatenate([edge_index[0], pad_idx]).reshape(NT, NCH, C)
    dstR = jnp.concatenate([edge_index[1], pad_idx]).reshape(NT, NCH, C)
    batch_p = jnp.concatenate(
        [batch, jnp.full((NP - N,), NGRAPH, jnp.int32)]).reshape(GRID, 1, RB)

    haug1, av1 = _dense_in(x_pad, W1, A1)
    acc1 = _sc_edge(haug1, av1[:, 0], av1[:, 1], srcR, dstR)
    haug2, av2 = _finish_dense(acc1, haug1, av1, b1.reshape(1, D_H), W2, A2)
    acc2 = _sc_edge(haug2, av2[:, 0], av2[:, 1], srcR, dstR)
    mu, sg = _finish_pool(acc2, haug2, av2, b2.reshape(1, D_H), batch_p,
                          W_mu, b_mu.reshape(1, -1),
                          W_sigma, b_sigma.reshape(1, -1))
    return (mu.reshape(NGRAPH, SEQ_OUT, OUT_DIM),
            sg.reshape(NGRAPH, SEQ_OUT, OUT_DIM))


def kernel(x, edge_index, batch, W1, a1_src, a1_dst, b1,
           W2, a2_src, a2_dst, b2, W_mu, b_mu, W_sigma, b_sigma):
    return _run(x, edge_index, batch, W1, a1_src, a1_dst, b1,
                W2, a2_src, a2_dst, b2, W_mu, b_mu, W_sigma, b_sigma)


# speedup vs baseline: 26.8897x; 1.0000x over previous
import functools

import jax
import jax.numpy as jnp
from jax import lax
from jax.experimental import pallas as pl
from jax.experimental.pallas import tpu as pltpu
from jax.experimental.pallas import tpu_sc as plsc

N = 10000
E = 320000
D_IN = 128
D_H = 64
NGRAPH = 32
SEQ_OUT = 12
OUT_DIM = 14

RB = 128
NP = 10240
GRID = NP // RB
DA = 80
NT = 32
C = 128
NCH = 80
ET = NCH * C
EP = NT * ET
PAD_NODE = N
STRIPE = NP // 16


def _dense_in_body(x_ref, w_ref, a_ref, haug_ref, av_ref):
    h = jnp.dot(x_ref[...], w_ref[...], preferred_element_type=jnp.float32)
    haug_ref[...] = jnp.concatenate(
        [h, jnp.ones((RB, 1), jnp.float32),
         jnp.zeros((RB, DA - D_H - 1), jnp.float32)], axis=1)
    av_ref[...] = jnp.dot(h, a_ref[...], preferred_element_type=jnp.float32)


_dense_in = pl.pallas_call(
    _dense_in_body,
    grid=(GRID,),
    in_specs=[
        pl.BlockSpec((RB, D_IN), lambda i: (i, 0)),
        pl.BlockSpec((D_IN, D_H), lambda i: (0, 0)),
        pl.BlockSpec((D_H, 128), lambda i: (0, 0)),
    ],
    out_specs=[
        pl.BlockSpec((RB, DA), lambda i: (i, 0)),
        pl.BlockSpec((RB, 128), lambda i: (i, 0)),
    ],
    out_shape=[
        jax.ShapeDtypeStruct((NP, DA), jnp.float32),
        jax.ShapeDtypeStruct((NP, 128), jnp.float32),
    ],
)


def _finish_block(acc_ref, haug_ref, av_ref, b_ref):
    acc = acc_ref[0] + acc_ref[1]
    h = haug_ref[...][:, :D_H]
    es = av_ref[...][:, 0:1] + av_ref[...][:, 1:2]
    ws = jnp.exp(jnp.where(es >= 0, es, es * 0.2))
    den = acc[:, D_H:D_H + 1] + ws
    out = (acc[:, :D_H] + ws * h) / den + b_ref[...]
    return jnp.maximum(out, 0.0)


def _finish_dense_body(acc_ref, haug_ref, av_ref, b_ref, w_ref, a_ref,
                       haug2_ref, av2_ref):
    out1 = _finish_block(acc_ref, haug_ref, av_ref, b_ref)
    h2 = jnp.dot(out1, w_ref[...], preferred_element_type=jnp.float32)
    haug2_ref[...] = jnp.concatenate(
        [h2, jnp.ones((RB, 1), jnp.float32),
         jnp.zeros((RB, DA - D_H - 1), jnp.float32)], axis=1)
    av2_ref[...] = jnp.dot(h2, a_ref[...], preferred_element_type=jnp.float32)


_finish_dense = pl.pallas_call(
    _finish_dense_body,
    grid=(GRID,),
    in_specs=[
        pl.BlockSpec((2, RB, DA), lambda i: (0, i, 0)),
        pl.BlockSpec((RB, DA), lambda i: (i, 0)),
        pl.BlockSpec((RB, 128), lambda i: (i, 0)),
        pl.BlockSpec((1, D_H), lambda i: (0, 0)),
        pl.BlockSpec((D_H, D_H), lambda i: (0, 0)),
        pl.BlockSpec((D_H, 128), lambda i: (0, 0)),
    ],
    out_specs=[
        pl.BlockSpec((RB, DA), lambda i: (i, 0)),
        pl.BlockSpec((RB, 128), lambda i: (i, 0)),
    ],
    out_shape=[
        jax.ShapeDtypeStruct((NP, DA), jnp.float32),
        jax.ShapeDtypeStruct((NP, 128), jnp.float32),
    ],
)


def _finish_pool_body(acc_ref, haug_ref, av_ref, b_ref, batch_ref,
                      wmu_ref, bmu_ref, wsg_ref, bsg_ref,
                      mu_ref, sg_ref, g_acc, c_acc):
    i = pl.program_id(0)

    @pl.when(i == 0)
    def _():
        g_acc[...] = jnp.zeros_like(g_acc)
        c_acc[...] = jnp.zeros_like(c_acc)

    out2 = _finish_block(acc_ref, haug_ref, av_ref, b_ref)
    bt = batch_ref[0]
    gids = lax.broadcasted_iota(jnp.int32, (NGRAPH, 1), 0)
    oht = (bt == gids).astype(jnp.float32)
    g_acc[...] += jnp.dot(oht, out2, preferred_element_type=jnp.float32)
    c_acc[...] += jnp.sum(oht, axis=1, keepdims=True)

    @pl.when(i == GRID - 1)
    def _():
        g = g_acc[...] / jnp.maximum(c_acc[...], 1.0)
        mu_ref[...] = (jnp.dot(g, wmu_ref[...],
                               preferred_element_type=jnp.float32)
                       + bmu_ref[...])
        z = (jnp.dot(g, wsg_ref[...], preferred_element_type=jnp.float32)
             + bsg_ref[...])
        sg_ref[...] = jnp.maximum(z, 0.0) + jnp.log1p(jnp.exp(-jnp.abs(z)))


_finish_pool = pl.pallas_call(
    _finish_pool_body,
    grid=(GRID,),
    in_specs=[
        pl.BlockSpec((2, RB, DA), lambda i: (0, i, 0)),
        pl.BlockSpec((RB, DA), lambda i: (i, 0)),
        pl.BlockSpec((RB, 128), lambda i: (i, 0)),
        pl.BlockSpec((1, D_H), lambda i: (0, 0)),
        pl.BlockSpec((1, 1, RB), lambda i: (i, 0, 0)),
        pl.BlockSpec((D_H, SEQ_OUT * OUT_DIM), lambda i: (0, 0)),
        pl.BlockSpec((1, SEQ_OUT * OUT_DIM), lambda i: (0, 0)),
        pl.BlockSpec((D_H, SEQ_OUT * OUT_DIM), lambda i: (0, 0)),
        pl.BlockSpec((1, SEQ_OUT * OUT_DIM), lambda i: (0, 0)),
    ],
    out_specs=[
        pl.BlockSpec((NGRAPH, SEQ_OUT * OUT_DIM), lambda i: (0, 0)),
        pl.BlockSpec((NGRAPH, SEQ_OUT * OUT_DIM), lambda i: (0, 0)),
    ],
    out_shape=[
        jax.ShapeDtypeStruct((NGRAPH, SEQ_OUT * OUT_DIM), jnp.float32),
        jax.ShapeDtypeStruct((NGRAPH, SEQ_OUT * OUT_DIM), jnp.float32),
    ],
    scratch_shapes=[
        pltpu.VMEM((NGRAPH, D_H), jnp.float32),
        pltpu.VMEM((NGRAPH, 1), jnp.float32),
    ],
)


def _sc_edge_body(h_hbm, as_hbm, ad_hbm, src_hbm, dst_hbm, acc_out,
                  acc_sh, as_v, ad_v, src_v, dst_v, w_v,
                  rows_a, rows_b, zrow_v, sem_a, sem_b):
    c = lax.axis_index("c")
    s = lax.axis_index("s")
    wid = s * 2 + c

    pltpu.sync_copy(as_hbm, as_v)
    pltpu.sync_copy(ad_hbm, ad_v)
    pltpu.sync_copy(src_hbm.at[wid], src_v)
    pltpu.sync_copy(dst_hbm.at[wid], dst_v)

    zv = jnp.zeros((16,), jnp.float32)

    def _zrow(r, carry):
        for q in range(DA // 16):
            zrow_v[r, pl.ds(q * 16, 16)] = zv
        return carry

    lax.fori_loop(0, C, _zrow, 0)
    for k in range(STRIPE // C):
        pltpu.sync_copy(zrow_v, acc_sh.at[pl.ds(s * STRIPE + k * C, C)])
    plsc.subcore_barrier()

    def _half(jj, rows, sem, rows_next, sem_next):
        @pl.when(jj + 1 < NCH)
        def _():
            pltpu.async_copy(h_hbm.at[src_v.at[jj + 1]], rows_next, sem_next)

        for g in range(C // 16):
            sv = src_v[jj, pl.ds(g * 16, 16)]
            dv = dst_v[jj, pl.ds(g * 16, 16)]
            e = plsc.load_gather(as_v, [sv]) + plsc.load_gather(ad_v, [dv])
            e = jnp.where(e >= 0, e, e * jnp.float32(0.2))
            w_v[pl.ds(g * 16, 16)] = jnp.exp(e)

        pltpu.make_async_copy(h_hbm.at[src_v.at[jj]], rows, sem).wait()

        def _scale(t, carry):
            wv = w_v[pl.ds(t * 16, 16)]
            for u in range(16):
                ei = t * 16 + u
                wsc = wv[u]
                for q in range(DA // 16):
                    rows[ei, pl.ds(q * 16, 16)] = (
                        rows[ei, pl.ds(q * 16, 16)] * wsc)
            return carry

        lax.fori_loop(0, C // 16, _scale, 0)
        pltpu.sync_copy(rows, acc_sh.at[dst_v.at[jj]], add=True)

    pltpu.async_copy(h_hbm.at[src_v.at[0]], rows_a, sem_a)

    def _pair(k, carry):
        _half(k * 2, rows_a, sem_a, rows_b, sem_b)
        _half(k * 2 + 1, rows_b, sem_b, rows_a, sem_a)
        return carry

    lax.fori_loop(0, NCH // 2, _pair, 0)
    plsc.subcore_barrier()

    for k in range(STRIPE // C):
        off = s * STRIPE + k * C
        pltpu.sync_copy(acc_sh.at[pl.ds(off, C)],
                        acc_out.at[c, pl.ds(off, C)])


_sc_edge = pl.kernel(
    _sc_edge_body,
    out_type=jax.ShapeDtypeStruct((2, NP, DA), jnp.float32),
    mesh=plsc.VectorSubcoreMesh(core_axis_name="c", subcore_axis_name="s",
                                num_cores=2, num_subcores=16),
    compiler_params=pltpu.CompilerParams(needs_layout_passes=False,
                                         use_tc_tiling_on_sc=False),
    scratch_types=[
        pltpu.VMEM_SHARED((NP, DA), jnp.float32),
        pltpu.VMEM((NP,), jnp.float32),
        pltpu.VMEM((NP,), jnp.float32),
        pltpu.VMEM((NCH, C), jnp.int32),
        pltpu.VMEM((NCH, C), jnp.int32),
        pltpu.VMEM((C,), jnp.float32),
        pltpu.VMEM((C, DA), jnp.float32),
        pltpu.VMEM((C, DA), jnp.float32),
        pltpu.VMEM((C, DA), jnp.float32),
        pltpu.SemaphoreType.DMA,
        pltpu.SemaphoreType.DMA,
    ],
)


@jax.jit
def _run(x, edge_index, batch, W1, a1_src, a1_dst, b1,
         W2, a2_src, a2_dst, b2, W_mu, b_mu, W_sigma, b_sigma):
    f32 = jnp.float32
    x_pad = jnp.pad(x, ((0, NP - N), (0, 0)))
    A1 = jnp.zeros((D_H, 128), f32).at[:, 0].set(a1_src).at[:, 1].set(a1_dst)
    A2 = jnp.zeros((D_H, 128), f32).at[:, 0].set(a2_src).at[:, 1].set(a2_dst)
    pad_src = jnp.full((EP - E,), PAD_NODE, jnp.int32)
    pad_dst = PAD_NODE + jnp.arange(EP - E, dtype=jnp.int32) % (NP - N)
    srcR = jnp.concatenate([edge_index[0], pad_src]).reshape(NT, NCH, C)
    dstR = jnp.concatenate([edge_index[1], pad_dst]).reshape(NT, NCH, C)
    batch_p = jnp.concatenate(
        [batch, jnp.full((NP - N,), NGRAPH, jnp.int32)]).reshape(GRID, 1, RB)

    haug1, av1 = _dense_in(x_pad, W1, A1)
    acc1 = _sc_edge(haug1, av1[:, 0], av1[:, 1], srcR, dstR)
    haug2, av2 = _finish_dense(acc1, haug1, av1, b1.reshape(1, D_H), W2, A2)
    acc2 = _sc_edge(haug2, av2[:, 0], av2[:, 1], srcR, dstR)
    mu, sg = _finish_pool(acc2, haug2, av2, b2.reshape(1, D_H), batch_p,
                          W_mu, b_mu.reshape(1, -1),
                          W_sigma, b_sigma.reshape(1, -1))
    return (mu.reshape(NGRAPH, SEQ_OUT, OUT_DIM),
            sg.reshape(NGRAPH, SEQ_OUT, OUT_DIM))


def kernel(x, edge_index, batch, W1, a1_src, a1_dst, b1,
           W2, a2_src, a2_dst, b2, W_mu, b_mu, W_sigma, b_sigma):
    return _run(x, edge_index, batch, W1, a1_src, a1_dst, b1,
                W2, a2_src, a2_dst, b2, W_mu, b_mu, W_sigma, b_sigma)
